# baseline (speedup 1.0000x reference)
import functools

import jax
import jax.numpy as jnp
from jax.experimental import pallas as pl
from jax.experimental.pallas import tpu as pltpu

NUM_EXPERTS = 8
N_EMBD = 1024
EXPERT_DIM = 2048
NUM_TOKENS = 4096
TOP_K = 2

NUM_ASSIGN = NUM_TOKENS * TOP_K
ROW_BLOCK = 128
NBLK = NUM_ASSIGN // ROW_BLOCK + NUM_EXPERTS
NP = NBLK * ROW_BLOCK

SUB = N_EMBD // 128
TOK_BLOCK = 128


def _moe_block_kernel(be_ref, tok_ref, x_ref, w1_ref, w2_ref, yg_ref, xg_scr):
    b = pl.program_id(0)
    base = b * ROW_BLOCK
    for r in range(ROW_BLOCK):
        t = tok_ref[base + r]
        xg_scr[pl.ds(r, 1)] = x_ref[pl.ds(t, 1)]
    xg = xg_scr[...].reshape(ROW_BLOCK, N_EMBD).astype(jnp.bfloat16)
    w1e = w1_ref[0]
    h = jax.lax.dot_general(xg, w1e, (((1,), (1,)), ((), ())),
                            preferred_element_type=jnp.float32)
    h = jnp.square(jnp.maximum(h, 0.0)).astype(jnp.bfloat16)
    w2e = w2_ref[0]
    y = jax.lax.dot_general(h, w2e, (((1,), (1,)), ((), ())),
                            preferred_element_type=jnp.float32)
    yg_ref[...] = y.reshape(ROW_BLOCK, SUB, 128)


def _combine_kernel(pos_ref, ew_ref, yg_ref, out_ref):
    b = pl.program_id(0)
    base = b * TOK_BLOCK
    for r in range(TOK_BLOCK):
        t = base + r
        p0 = pos_ref[2 * t]
        p1 = pos_ref[2 * t + 1]
        w0 = ew_ref[2 * t]
        w1 = ew_ref[2 * t + 1]
        out_ref[pl.ds(r, 1)] = (w0 * yg_ref[pl.ds(p0, 1)]
                                + w1 * yg_ref[pl.ds(p1, 1)])


@jax.jit
def _run(x, expert_indices, expert_weights, w1, w2):
    e = expert_indices.reshape(-1).astype(jnp.int32)
    ew = expert_weights.reshape(-1).astype(jnp.float32)

    oh = (e[:, None] == jnp.arange(NUM_EXPERTS, dtype=jnp.int32)).astype(jnp.int32)
    cum = jnp.cumsum(oh, axis=0)
    counts = cum[-1]
    rank = jnp.take_along_axis(cum, e[:, None], axis=1)[:, 0] - 1
    padded = ((counts + ROW_BLOCK - 1) // ROW_BLOCK) * ROW_BLOCK
    starts = jnp.concatenate([jnp.zeros(1, jnp.int32), jnp.cumsum(padded)]).astype(jnp.int32)
    pos = starts[e] + rank

    tok = jnp.arange(NUM_ASSIGN, dtype=jnp.int32) // TOP_K
    src_token = jnp.zeros((NP,), jnp.int32).at[pos].set(tok)

    blk_start = jnp.arange(NBLK, dtype=jnp.int32) * ROW_BLOCK
    block_expert = jnp.clip(
        jnp.searchsorted(starts[1:], blk_start, side='right').astype(jnp.int32),
        0, NUM_EXPERTS - 1)

    x3 = x.reshape(NUM_TOKENS, SUB, 128)
    w1b = w1.astype(jnp.bfloat16)
    w2b = w2.astype(jnp.bfloat16)

    yg = pl.pallas_call(
        _moe_block_kernel,
        grid_spec=pltpu.PrefetchScalarGridSpec(
            num_scalar_prefetch=2,
            grid=(NBLK,),
            in_specs=[
                pl.BlockSpec((NUM_TOKENS, SUB, 128), lambda b, be, tk: (0, 0, 0)),
                pl.BlockSpec((1, EXPERT_DIM, N_EMBD), lambda b, be, tk: (be[b], 0, 0)),
                pl.BlockSpec((1, N_EMBD, EXPERT_DIM), lambda b, be, tk: (be[b], 0, 0)),
            ],
            out_specs=pl.BlockSpec((ROW_BLOCK, SUB, 128), lambda b, be, tk: (b, 0, 0)),
            scratch_shapes=[pltpu.VMEM((ROW_BLOCK, SUB, 128), jnp.float32)],
        ),
        out_shape=jax.ShapeDtypeStruct((NP, SUB, 128), jnp.float32),
        compiler_params=pltpu.CompilerParams(
            dimension_semantics=("arbitrary",),
        ),
    )(block_expert, src_token, x3, w1b, w2b)

    out3 = pl.pallas_call(
        _combine_kernel,
        grid_spec=pltpu.PrefetchScalarGridSpec(
            num_scalar_prefetch=2,
            grid=(NUM_TOKENS // TOK_BLOCK,),
            in_specs=[
                pl.BlockSpec((NP, SUB, 128), lambda b, ps, w: (0, 0, 0)),
            ],
            out_specs=pl.BlockSpec((TOK_BLOCK, SUB, 128), lambda b, ps, w: (b, 0, 0)),
        ),
        out_shape=jax.ShapeDtypeStruct((NUM_TOKENS, SUB, 128), jnp.float32),
        compiler_params=pltpu.CompilerParams(
            dimension_semantics=("arbitrary",),
        ),
    )(pos, ew, yg)

    return out3.reshape(NUM_TOKENS, N_EMBD)


def kernel(x, expert_indices, expert_weights, w1, w2):
    return _run(x, expert_indices, expert_weights, w1, w2)


# speedup vs baseline: 1.1606x; 1.1606x over previous
import functools

import jax
import jax.numpy as jnp
from jax.experimental import pallas as pl
from jax.experimental.pallas import tpu as pltpu

NUM_EXPERTS = 8
N_EMBD = 1024
EXPERT_DIM = 2048
NUM_TOKENS = 4096
TOP_K = 2

NUM_ASSIGN = NUM_TOKENS * TOP_K
ROW_BLOCK = 128
NBLK = NUM_ASSIGN // ROW_BLOCK + NUM_EXPERTS
NP = NBLK * ROW_BLOCK

SUB = N_EMBD // 128
TOK_BLOCK = 128


def _moe_block_kernel(be_ref, tok_ref, act_ref, x_ref, w1_ref, w2_ref, yg_ref,
                      xg_scr):
    b = pl.program_id(0)

    @pl.when(act_ref[b] > 0)
    def _():
        base = b * ROW_BLOCK
        for r in range(ROW_BLOCK):
            t = tok_ref[base + r]
            xg_scr[pl.ds(r, 1)] = x_ref[pl.ds(t, 1)]
        xg = xg_scr[...].reshape(ROW_BLOCK, N_EMBD)
        w1e = w1_ref[0]
        h = jax.lax.dot_general(xg, w1e, (((1,), (1,)), ((), ())),
                                preferred_element_type=jnp.float32)
        h = jnp.square(jnp.maximum(h, 0.0))
        w2e = w2_ref[0]
        y = jax.lax.dot_general(h, w2e, (((1,), (1,)), ((), ())),
                                preferred_element_type=jnp.float32)
        yg_ref[...] = y.reshape(ROW_BLOCK, SUB, 128)


def _combine_kernel(pos_ref, ew_ref, yg_ref, out_ref):
    b = pl.program_id(0)
    base = b * TOK_BLOCK
    for r in range(TOK_BLOCK):
        t = base + r
        p0 = pos_ref[2 * t]
        p1 = pos_ref[2 * t + 1]
        w0 = ew_ref[2 * t]
        w1 = ew_ref[2 * t + 1]
        out_ref[pl.ds(r, 1)] = (w0 * yg_ref[pl.ds(p0, 1)]
                                + w1 * yg_ref[pl.ds(p1, 1)])


@jax.jit
def _run(x, expert_indices, expert_weights, w1, w2):
    e = expert_indices.reshape(-1).astype(jnp.int32)
    ew = expert_weights.reshape(-1).astype(jnp.float32)

    oh = (e[:, None] == jnp.arange(NUM_EXPERTS, dtype=jnp.int32)).astype(jnp.int32)
    cum = jnp.cumsum(oh, axis=0)
    counts = cum[-1]
    rank = jnp.take_along_axis(cum, e[:, None], axis=1)[:, 0] - 1
    padded = ((counts + ROW_BLOCK - 1) // ROW_BLOCK) * ROW_BLOCK
    starts = jnp.concatenate([jnp.zeros(1, jnp.int32), jnp.cumsum(padded)]).astype(jnp.int32)
    pos = starts[e] + rank

    tok = jnp.arange(NUM_ASSIGN, dtype=jnp.int32) // TOP_K
    src_token = jnp.zeros((NP,), jnp.int32).at[pos].set(tok)

    blk_start = jnp.arange(NBLK, dtype=jnp.int32) * ROW_BLOCK
    block_expert = jnp.clip(
        jnp.searchsorted(starts[1:], blk_start, side='right').astype(jnp.int32),
        0, NUM_EXPERTS - 1)
    active = (blk_start < starts[NUM_EXPERTS]).astype(jnp.int32)

    x3 = x.reshape(NUM_TOKENS, SUB, 128)

    yg = pl.pallas_call(
        _moe_block_kernel,
        grid_spec=pltpu.PrefetchScalarGridSpec(
            num_scalar_prefetch=3,
            grid=(NBLK,),
            in_specs=[
                pl.BlockSpec((NUM_TOKENS, SUB, 128), lambda b, *_: (0, 0, 0)),
                pl.BlockSpec((1, EXPERT_DIM, N_EMBD), lambda b, be, *_: (be[b], 0, 0)),
                pl.BlockSpec((1, N_EMBD, EXPERT_DIM), lambda b, be, *_: (be[b], 0, 0)),
            ],
            out_specs=pl.BlockSpec((ROW_BLOCK, SUB, 128), lambda b, *_: (b, 0, 0)),
            scratch_shapes=[pltpu.VMEM((ROW_BLOCK, SUB, 128), jnp.float32)],
        ),
        out_shape=jax.ShapeDtypeStruct((NP, SUB, 128), jnp.float32),
        compiler_params=pltpu.CompilerParams(
            dimension_semantics=("arbitrary",),
            vmem_limit_bytes=100 * 1024 * 1024,
        ),
    )(block_expert, src_token, active, x3, w1, w2)

    out3 = pl.pallas_call(
        _combine_kernel,
        grid_spec=pltpu.PrefetchScalarGridSpec(
            num_scalar_prefetch=2,
            grid=(NUM_TOKENS // TOK_BLOCK,),
            in_specs=[
                pl.BlockSpec((NP, SUB, 128), lambda b, *_: (0, 0, 0)),
            ],
            out_specs=pl.BlockSpec((TOK_BLOCK, SUB, 128), lambda b, *_: (b, 0, 0)),
        ),
        out_shape=jax.ShapeDtypeStruct((NUM_TOKENS, SUB, 128), jnp.float32),
        compiler_params=pltpu.CompilerParams(
            dimension_semantics=("arbitrary",),
            vmem_limit_bytes=100 * 1024 * 1024,
        ),
    )(pos, ew, yg)

    return out3.reshape(NUM_TOKENS, N_EMBD)


def kernel(x, expert_indices, expert_weights, w1, w2):
    return _run(x, expert_indices, expert_weights, w1, w2)


# speedup vs baseline: 1.1657x; 1.0044x over previous
import functools

import jax
import jax.numpy as jnp
from jax.experimental import pallas as pl
from jax.experimental.pallas import tpu as pltpu

NUM_EXPERTS = 8
N_EMBD = 1024
EXPERT_DIM = 2048
NUM_TOKENS = 4096
TOP_K = 2

NUM_ASSIGN = NUM_TOKENS * TOP_K
ROW_BLOCK = 128
NBLK = NUM_ASSIGN // ROW_BLOCK + NUM_EXPERTS
NP = NBLK * ROW_BLOCK

SUB = N_EMBD // 128
TOK_BLOCK = 128


def _moe_block_kernel(be_ref, tok_ref, act_ref, first_ref, par_ref, pe_ref,
                      pv_ref, x_ref, w1_ref, w2_ref, yg_ref,
                      xg_scr, w1_buf, w2_buf, sems):
    b = pl.program_id(0)

    @pl.when(b == 0)
    def _():
        e0 = be_ref[0]
        pltpu.make_async_copy(w1_ref.at[e0], w1_buf.at[0], sems.at[0, 0]).start()
        pltpu.make_async_copy(w2_ref.at[e0], w2_buf.at[0], sems.at[0, 1]).start()

    @pl.when(first_ref[b] == 1)
    def _():
        par = par_ref[b]

        @pl.when(pv_ref[b] == 1)
        def _():
            nslot = 1 - par
            pe = pe_ref[b]
            pltpu.make_async_copy(w1_ref.at[pe], w1_buf.at[nslot],
                                  sems.at[nslot, 0]).start()
            pltpu.make_async_copy(w2_ref.at[pe], w2_buf.at[nslot],
                                  sems.at[nslot, 1]).start()

        e = be_ref[b]
        pltpu.make_async_copy(w1_ref.at[e], w1_buf.at[par], sems.at[par, 0]).wait()
        pltpu.make_async_copy(w2_ref.at[e], w2_buf.at[par], sems.at[par, 1]).wait()

    @pl.when(act_ref[b] > 0)
    def _():
        base = b * ROW_BLOCK
        for r in range(ROW_BLOCK):
            t = tok_ref[base + r]
            xg_scr[pl.ds(r, 1)] = x_ref[pl.ds(t, 1)]
        xg = xg_scr[...].reshape(ROW_BLOCK, N_EMBD)
        par = par_ref[b]
        w1e = w1_buf[par]
        h = jax.lax.dot_general(xg, w1e, (((1,), (1,)), ((), ())),
                                preferred_element_type=jnp.float32)
        h = jnp.square(jnp.maximum(h, 0.0))
        w2e = w2_buf[par]
        y = jax.lax.dot_general(h, w2e, (((1,), (1,)), ((), ())),
                                preferred_element_type=jnp.float32)
        yg_ref[...] = y.reshape(ROW_BLOCK, SUB, 128)


def _combine_kernel(pos_ref, ew_ref, yg_ref, out_ref):
    b = pl.program_id(0)
    base = b * TOK_BLOCK
    for r in range(TOK_BLOCK):
        t = base + r
        p0 = pos_ref[2 * t]
        p1 = pos_ref[2 * t + 1]
        w0 = ew_ref[2 * t]
        w1 = ew_ref[2 * t + 1]
        out_ref[pl.ds(r, 1)] = (w0 * yg_ref[pl.ds(p0, 1)]
                                + w1 * yg_ref[pl.ds(p1, 1)])


@jax.jit
def _run(x, expert_indices, expert_weights, w1, w2):
    e = expert_indices.reshape(-1).astype(jnp.int32)
    ew = expert_weights.reshape(-1).astype(jnp.float32)

    oh = (e[:, None] == jnp.arange(NUM_EXPERTS, dtype=jnp.int32)).astype(jnp.int32)
    cum = jnp.cumsum(oh, axis=0)
    counts = cum[-1]
    rank = jnp.take_along_axis(cum, e[:, None], axis=1)[:, 0] - 1
    padded = ((counts + ROW_BLOCK - 1) // ROW_BLOCK) * ROW_BLOCK
    starts = jnp.concatenate([jnp.zeros(1, jnp.int32), jnp.cumsum(padded)]).astype(jnp.int32)
    pos = starts[e] + rank

    tok = jnp.arange(NUM_ASSIGN, dtype=jnp.int32) // TOP_K
    src_token = jnp.zeros((NP,), jnp.int32).at[pos].set(tok)

    blk_start = jnp.arange(NBLK, dtype=jnp.int32) * ROW_BLOCK
    block_expert = jnp.clip(
        jnp.searchsorted(starts[1:], blk_start, side='right').astype(jnp.int32),
        0, NUM_EXPERTS - 1)
    active = (blk_start < starts[NUM_EXPERTS]).astype(jnp.int32)

    prev_be = jnp.concatenate([jnp.array([-1], jnp.int32), block_expert[:-1]])
    first = ((block_expert != prev_be) & (active == 1)).astype(jnp.int32)
    run_idx = jnp.cumsum(first) - 1
    parity = (run_idx % 2).astype(jnp.int32)
    bidx = jnp.arange(NBLK, dtype=jnp.int32)
    nf = jnp.flip(jax.lax.cummin(jnp.flip(
        jnp.where(first == 1, bidx, NBLK)))).astype(jnp.int32)
    nxt_first = jnp.concatenate([nf[1:], jnp.array([NBLK], jnp.int32)])
    pv = (nxt_first < NBLK).astype(jnp.int32)
    pe = block_expert[jnp.minimum(nxt_first, NBLK - 1)]

    x3 = x.reshape(NUM_TOKENS, SUB, 128)

    yg = pl.pallas_call(
        _moe_block_kernel,
        grid_spec=pltpu.PrefetchScalarGridSpec(
            num_scalar_prefetch=7,
            grid=(NBLK,),
            in_specs=[
                pl.BlockSpec((NUM_TOKENS, SUB, 128), lambda b, *_: (0, 0, 0)),
                pl.BlockSpec(memory_space=pl.ANY),
                pl.BlockSpec(memory_space=pl.ANY),
            ],
            out_specs=pl.BlockSpec((ROW_BLOCK, SUB, 128), lambda b, *_: (b, 0, 0)),
            scratch_shapes=[
                pltpu.VMEM((ROW_BLOCK, SUB, 128), jnp.float32),
                pltpu.VMEM((2, EXPERT_DIM, N_EMBD), jnp.float32),
                pltpu.VMEM((2, N_EMBD, EXPERT_DIM), jnp.float32),
                pltpu.SemaphoreType.DMA((2, 2)),
            ],
        ),
        out_shape=jax.ShapeDtypeStruct((NP, SUB, 128), jnp.float32),
        compiler_params=pltpu.CompilerParams(
            dimension_semantics=("arbitrary",),
            vmem_limit_bytes=100 * 1024 * 1024,
        ),
    )(block_expert, src_token, active, first, parity, pe, pv, x3, w1, w2)

    out3 = pl.pallas_call(
        _combine_kernel,
        grid_spec=pltpu.PrefetchScalarGridSpec(
            num_scalar_prefetch=2,
            grid=(NUM_TOKENS // TOK_BLOCK,),
            in_specs=[
                pl.BlockSpec((NP, SUB, 128), lambda b, *_: (0, 0, 0)),
            ],
            out_specs=pl.BlockSpec((TOK_BLOCK, SUB, 128), lambda b, *_: (b, 0, 0)),
        ),
        out_shape=jax.ShapeDtypeStruct((NUM_TOKENS, SUB, 128), jnp.float32),
        compiler_params=pltpu.CompilerParams(
            dimension_semantics=("arbitrary",),
            vmem_limit_bytes=100 * 1024 * 1024,
        ),
    )(pos, ew, yg)

    return out3.reshape(NUM_TOKENS, N_EMBD)


def kernel(x, expert_indices, expert_weights, w1, w2):
    return _run(x, expert_indices, expert_weights, w1, w2)


# speedup vs baseline: 1.3406x; 1.1501x over previous
import functools

import jax
import jax.numpy as jnp
from jax import lax
from jax.experimental import pallas as pl
from jax.experimental.pallas import tpu as pltpu
from jax.experimental.pallas import tpu_sc as plsc

NUM_EXPERTS = 8
N_EMBD = 1024
EXPERT_DIM = 2048
NUM_TOKENS = 4096
TOP_K = 2

NUM_ASSIGN = NUM_TOKENS * TOP_K
ROW_BLOCK = 128
NBLK = NUM_ASSIGN // ROW_BLOCK + NUM_EXPERTS
NP = NBLK * ROW_BLOCK

SUB = N_EMBD // 128
TOK_BLOCK = 128


NV = NUM_ASSIGN // 16
NPV = NP // 16


def _zed16():
    return jnp.zeros((16,), jnp.int32)


def _sc_meta_kernel(e_hbm, cnt_hbm, pos_hbm, srctok_hbm,
                    e_pad, rank_v, pos_v, srctok_v, cnt_v, starts_v, pfx_v):
    cid = lax.axis_index("c")
    sid = lax.axis_index("s")

    @pl.when(jnp.logical_and(cid == 0, sid == 0))
    def _():
        e_pad[pl.ds(0, 16)] = jnp.full((16,), -1, jnp.int32)
        pltpu.sync_copy(e_hbm, e_pad.at[pl.ds(16, NUM_ASSIGN)])
        cnt_v[...] = _zed16()
        ones = jnp.ones((16,), jnp.int32)
        lanes = lax.iota(jnp.int32, 16)

        def body1(i, carry):
            v = e_pad[pl.ds(16 + i * 16, 16)]
            base = plsc.load_gather(cnt_v, [v])
            intra = _zed16()
            for k in range(1, 16):
                w = e_pad[pl.ds(16 + i * 16 - k, 16)]
                m = jnp.logical_and(w == v, lanes >= k)
                intra = intra + m.astype(jnp.int32)
            rank_v[pl.ds(i * 16, 16)] = base + intra
            plsc.addupdate_scatter(cnt_v, [v], ones)
            return carry

        lax.fori_loop(0, NV, body1, 0)

        cnt = cnt_v[...]
        padded = jnp.where(
            lanes < NUM_EXPERTS,
            ((cnt + (ROW_BLOCK - 1)) // ROW_BLOCK) * ROW_BLOCK,
            0)
        pfx_v[pl.ds(0, 16)] = _zed16()
        pfx_v[pl.ds(16, 16)] = padded
        incl = padded
        for k in (1, 2, 4, 8):
            shifted = pfx_v[pl.ds(16 - k, 16)]
            incl = incl + shifted
            pfx_v[pl.ds(16, 16)] = incl
        starts_v[...] = incl - padded
        cnt_v[...] = jnp.where(lanes < NUM_EXPERTS, cnt, incl)

        def bodyz(j, carry):
            srctok_v[pl.ds(j * 16, 16)] = _zed16()
            return carry

        lax.fori_loop(0, NPV, bodyz, 0)

        def body2(i, carry):
            v = e_pad[pl.ds(16 + i * 16, 16)]
            r = rank_v[pl.ds(i * 16, 16)]
            s = plsc.load_gather(starts_v, [v])
            p = s + r
            pos_v[pl.ds(i * 16, 16)] = p
            tokv = jax.lax.shift_right_logical(lanes + i * 16, 1)
            plsc.store_scatter(srctok_v, [p], tokv)
            return carry

        lax.fori_loop(0, NV, body2, 0)

        pltpu.sync_copy(cnt_v, cnt_hbm)
        pltpu.sync_copy(pos_v, pos_hbm)
        pltpu.sync_copy(srctok_v, srctok_hbm)


def _sc_routing_metadata(e_flat):
    mesh = plsc.VectorSubcoreMesh(core_axis_name="c", subcore_axis_name="s")
    k = pl.kernel(
        _sc_meta_kernel,
        out_type=(
            jax.ShapeDtypeStruct((16,), jnp.int32),
            jax.ShapeDtypeStruct((NUM_ASSIGN,), jnp.int32),
            jax.ShapeDtypeStruct((NP,), jnp.int32),
        ),
        mesh=mesh,
        compiler_params=pltpu.CompilerParams(needs_layout_passes=False),
        scratch_types=[
            pltpu.VMEM((16 + NUM_ASSIGN,), jnp.int32),
            pltpu.VMEM((NUM_ASSIGN,), jnp.int32),
            pltpu.VMEM((NUM_ASSIGN,), jnp.int32),
            pltpu.VMEM((NP,), jnp.int32),
            pltpu.VMEM((16,), jnp.int32),
            pltpu.VMEM((16,), jnp.int32),
            pltpu.VMEM((32,), jnp.int32),
        ],
    )
    return k(e_flat)


def _moe_block_kernel(be_ref, tok_ref, act_ref, first_ref, par_ref, pe_ref,
                      pv_ref, x_ref, w1_ref, w2_ref, yg_ref,
                      xg_scr, w1_buf, w2_buf, sems):
    b = pl.program_id(0)

    @pl.when(b == 0)
    def _():
        e0 = be_ref[0]
        pltpu.make_async_copy(w1_ref.at[e0], w1_buf.at[0], sems.at[0, 0]).start()
        pltpu.make_async_copy(w2_ref.at[e0], w2_buf.at[0], sems.at[0, 1]).start()

    @pl.when(first_ref[b] == 1)
    def _():
        par = par_ref[b]

        @pl.when(pv_ref[b] == 1)
        def _():
            nslot = 1 - par
            pe = pe_ref[b]
            pltpu.make_async_copy(w1_ref.at[pe], w1_buf.at[nslot],
                                  sems.at[nslot, 0]).start()
            pltpu.make_async_copy(w2_ref.at[pe], w2_buf.at[nslot],
                                  sems.at[nslot, 1]).start()

        e = be_ref[b]
        pltpu.make_async_copy(w1_ref.at[e], w1_buf.at[par], sems.at[par, 0]).wait()
        pltpu.make_async_copy(w2_ref.at[e], w2_buf.at[par], sems.at[par, 1]).wait()

    @pl.when(act_ref[b] > 0)
    def _():
        base = b * ROW_BLOCK
        for r in range(ROW_BLOCK):
            t = tok_ref[base + r]
            xg_scr[pl.ds(r, 1)] = x_ref[pl.ds(t, 1)]
        xg = xg_scr[...].reshape(ROW_BLOCK, N_EMBD).astype(jnp.bfloat16)
        par = par_ref[b]
        w1e = w1_buf[par]
        h = jax.lax.dot_general(xg, w1e, (((1,), (0,)), ((), ())),
                                preferred_element_type=jnp.float32)
        h = jnp.square(jnp.maximum(h, 0.0)).astype(jnp.bfloat16)
        w2e = w2_buf[par]
        y = jax.lax.dot_general(h, w2e, (((1,), (0,)), ((), ())),
                                preferred_element_type=jnp.float32)
        yg_ref[...] = y.reshape(ROW_BLOCK, SUB, 128)


def _combine_kernel(pos_ref, ew_ref, yg_ref, out_ref):
    b = pl.program_id(0)
    base = b * TOK_BLOCK
    for r in range(TOK_BLOCK):
        t = base + r
        p0 = pos_ref[2 * t]
        p1 = pos_ref[2 * t + 1]
        w0 = ew_ref[2 * t]
        w1 = ew_ref[2 * t + 1]
        out_ref[pl.ds(r, 1)] = (w0 * yg_ref[pl.ds(p0, 1)]
                                + w1 * yg_ref[pl.ds(p1, 1)])


@jax.jit
def _run(x, expert_indices, expert_weights, w1, w2):
    e = expert_indices.reshape(-1).astype(jnp.int32)
    ew = expert_weights.reshape(-1).astype(jnp.float32)

    cnt16, pos, src_token = _sc_routing_metadata(e)
    counts = cnt16[:NUM_EXPERTS]
    padded = ((counts + ROW_BLOCK - 1) // ROW_BLOCK) * ROW_BLOCK
    starts = jnp.concatenate([jnp.zeros(1, jnp.int32), jnp.cumsum(padded)]).astype(jnp.int32)

    blk_start = jnp.arange(NBLK, dtype=jnp.int32) * ROW_BLOCK
    block_expert = jnp.clip(
        jnp.searchsorted(starts[1:], blk_start, side='right').astype(jnp.int32),
        0, NUM_EXPERTS - 1)
    active = (blk_start < starts[NUM_EXPERTS]).astype(jnp.int32)

    prev_be = jnp.concatenate([jnp.array([-1], jnp.int32), block_expert[:-1]])
    first = ((block_expert != prev_be) & (active == 1)).astype(jnp.int32)
    run_idx = jnp.cumsum(first) - 1
    parity = (run_idx % 2).astype(jnp.int32)
    bidx = jnp.arange(NBLK, dtype=jnp.int32)
    nf = jnp.flip(jax.lax.cummin(jnp.flip(
        jnp.where(first == 1, bidx, NBLK)))).astype(jnp.int32)
    nxt_first = jnp.concatenate([nf[1:], jnp.array([NBLK], jnp.int32)])
    pv = (nxt_first < NBLK).astype(jnp.int32)
    pe = block_expert[jnp.minimum(nxt_first, NBLK - 1)]

    x3 = x.reshape(NUM_TOKENS, SUB, 128)

    yg = pl.pallas_call(
        _moe_block_kernel,
        grid_spec=pltpu.PrefetchScalarGridSpec(
            num_scalar_prefetch=7,
            grid=(NBLK,),
            in_specs=[
                pl.BlockSpec((NUM_TOKENS, SUB, 128), lambda b, *_: (0, 0, 0)),
                pl.BlockSpec(memory_space=pl.ANY),
                pl.BlockSpec(memory_space=pl.ANY),
            ],
            out_specs=pl.BlockSpec((ROW_BLOCK, SUB, 128), lambda b, *_: (b, 0, 0)),
            scratch_shapes=[
                pltpu.VMEM((ROW_BLOCK, SUB, 128), jnp.float32),
                pltpu.VMEM((2, N_EMBD, EXPERT_DIM), jnp.bfloat16),
                pltpu.VMEM((2, EXPERT_DIM, N_EMBD), jnp.bfloat16),
                pltpu.SemaphoreType.DMA((2, 2)),
            ],
        ),
        out_shape=jax.ShapeDtypeStruct((NP, SUB, 128), jnp.float32),
        compiler_params=pltpu.CompilerParams(
            dimension_semantics=("arbitrary",),
            vmem_limit_bytes=100 * 1024 * 1024,
        ),
    )(block_expert, src_token, active, first, parity, pe, pv, x3, jnp.swapaxes(w1, 1, 2).astype(jnp.bfloat16), jnp.swapaxes(w2, 1, 2).astype(jnp.bfloat16))

    out3 = pl.pallas_call(
        _combine_kernel,
        grid_spec=pltpu.PrefetchScalarGridSpec(
            num_scalar_prefetch=2,
            grid=(NUM_TOKENS // TOK_BLOCK,),
            in_specs=[
                pl.BlockSpec((NP, SUB, 128), lambda b, *_: (0, 0, 0)),
            ],
            out_specs=pl.BlockSpec((TOK_BLOCK, SUB, 128), lambda b, *_: (b, 0, 0)),
        ),
        out_shape=jax.ShapeDtypeStruct((NUM_TOKENS, SUB, 128), jnp.float32),
        compiler_params=pltpu.CompilerParams(
            dimension_semantics=("arbitrary",),
            vmem_limit_bytes=100 * 1024 * 1024,
        ),
    )(pos, ew, yg)

    return out3.reshape(NUM_TOKENS, N_EMBD)


def kernel(x, expert_indices, expert_weights, w1, w2):
    return _run(x, expert_indices, expert_weights, w1, w2)
